# baseline (speedup 1.0000x reference)
import functools

import jax
import jax.numpy as jnp
from jax import lax
from jax.experimental import pallas as pl
from jax.experimental.pallas import tpu as pltpu
from jax.experimental.pallas import tpu_sc as plsc

VOCAB = 1000000
DIM = 64
B = 4096
L = 50
TOTAL = B * L

_info = plsc.get_sparse_core_info()
NC = _info.num_cores
NS = _info.num_subcores
NW = NC * NS
PER_W = TOTAL // NW
CHUNK = 128
NCHUNK = PER_W // CHUNK

_mesh = plsc.VectorSubcoreMesh(core_axis_name="c", subcore_axis_name="s")


@functools.partial(
    pl.kernel,
    mesh=_mesh,
    compiler_params=pltpu.CompilerParams(use_tc_tiling_on_sc=False),
    out_type=jax.ShapeDtypeStruct((TOTAL, DIM), jnp.float32),
    scratch_types=[
        pltpu.VMEM((CHUNK,), jnp.int32),
        pltpu.VMEM((CHUNK, DIM), jnp.float32),
        pltpu.SemaphoreType.DMA,
    ],
)
def _gather(table_hbm, idx_hbm, out_hbm, idx_c, rows_v, sem):
    wid = lax.axis_index("s") * NC + lax.axis_index("c")
    base = wid * PER_W
    def body(i, carry):
        pltpu.sync_copy(idx_hbm.at[wid, i], idx_c)
        pltpu.async_copy(table_hbm.at[idx_c], rows_v, sem).wait()
        pltpu.sync_copy(rows_v, out_hbm.at[pl.ds(base + i * CHUNK, CHUNK)])
        return carry

    lax.fori_loop(0, NCHUNK, body, 0)


def kernel(inputs, table):
    idx = inputs.reshape(NW, NCHUNK, CHUNK).astype(jnp.int32)
    out = _gather(table, idx)
    return out.reshape(B, L, DIM)


# speedup vs baseline: 1.0763x; 1.0763x over previous
import functools

import jax
import jax.numpy as jnp
from jax import lax
from jax.experimental import pallas as pl
from jax.experimental.pallas import tpu as pltpu
from jax.experimental.pallas import tpu_sc as plsc

VOCAB = 1000000
DIM = 64
B = 4096
L = 50
TOTAL = B * L

_info = plsc.get_sparse_core_info()
NC = _info.num_cores
NS = _info.num_subcores
NW = NC * NS
PER_W = TOTAL // NW
CHUNK = 800
NCHUNK = PER_W // CHUNK

_mesh = plsc.VectorSubcoreMesh(core_axis_name="c", subcore_axis_name="s")


@functools.partial(
    pl.kernel,
    mesh=_mesh,
    compiler_params=pltpu.CompilerParams(use_tc_tiling_on_sc=False),
    out_type=jax.ShapeDtypeStruct((TOTAL, DIM), jnp.float32),
    scratch_types=[
        pltpu.VMEM((NCHUNK, CHUNK), jnp.int32),
        pltpu.VMEM((2, CHUNK, DIM), jnp.float32),
        pltpu.SemaphoreType.DMA,
        pltpu.SemaphoreType.DMA,
        pltpu.SemaphoreType.DMA,
        pltpu.SemaphoreType.DMA,
    ],
)
def _gather(table_hbm, idx_hbm, out_hbm, idx_v, rows_v, gsem0, gsem1, osem0, osem1):
    wid = lax.axis_index("s") * NC + lax.axis_index("c")
    base = wid * PER_W
    gsems = (gsem0, gsem1)
    osems = (osem0, osem1)
    pltpu.sync_copy(idx_hbm.at[wid], idx_v)
    pend_g = [None, None]
    pend_o = [None, None]
    pend_g[0] = pltpu.async_copy(table_hbm.at[idx_v.at[0]], rows_v.at[0], gsems[0])
    for i in range(NCHUNK):
        b = i % 2
        nb = (i + 1) % 2
        if i + 1 < NCHUNK:
            if pend_o[nb] is not None:
                pend_o[nb].wait()
            pend_g[nb] = pltpu.async_copy(
                table_hbm.at[idx_v.at[i + 1]], rows_v.at[nb], gsems[nb])
        pend_g[b].wait()
        pend_o[b] = pltpu.async_copy(
            rows_v.at[b], out_hbm.at[pl.ds(base + i * CHUNK, CHUNK)], osems[b])
    pend_o[0].wait()
    pend_o[1].wait()


def kernel(inputs, table):
    idx = inputs.reshape(NW, NCHUNK, CHUNK).astype(jnp.int32)
    out = _gather(table, idx)
    return out.reshape(B, L, DIM)


# speedup vs baseline: 1.0864x; 1.0094x over previous
import functools

import jax
import jax.numpy as jnp
from jax import lax
from jax.experimental import pallas as pl
from jax.experimental.pallas import tpu as pltpu
from jax.experimental.pallas import tpu_sc as plsc

VOCAB = 1000000
DIM = 64
PDIM = 128
B = 4096
L = 50
TOTAL = B * L

_info = plsc.get_sparse_core_info()
NC = _info.num_cores
NS = _info.num_subcores
NW = NC * NS
PER_W = TOTAL // NW
CHUNK = 400
NCHUNK = PER_W // CHUNK

_mesh = plsc.VectorSubcoreMesh(core_axis_name="c", subcore_axis_name="s")


@functools.partial(
    pl.kernel,
    mesh=_mesh,
    compiler_params=pltpu.CompilerParams(use_tc_tiling_on_sc=False),
    out_type=jax.ShapeDtypeStruct((TOTAL, PDIM), jnp.float32),
    scratch_types=[
        pltpu.VMEM((NCHUNK, CHUNK), jnp.int32),
        pltpu.VMEM((2, CHUNK, PDIM), jnp.float32),
        pltpu.SemaphoreType.DMA,
        pltpu.SemaphoreType.DMA,
        pltpu.SemaphoreType.DMA,
        pltpu.SemaphoreType.DMA,
    ],
)
def _gather(table_hbm, idx_hbm, out_hbm, idx_v, rows_v, gsem0, gsem1, osem0, osem1):
    wid = lax.axis_index("s") * NC + lax.axis_index("c")
    base = wid * PER_W
    gsems = (gsem0, gsem1)
    osems = (osem0, osem1)
    pltpu.sync_copy(idx_hbm.at[wid], idx_v)
    pend_g = [None, None]
    pend_o = [None, None]
    pend_g[0] = pltpu.async_copy(table_hbm.at[idx_v.at[0]], rows_v.at[0], gsems[0])
    for i in range(NCHUNK):
        b = i % 2
        nb = (i + 1) % 2
        if i + 1 < NCHUNK:
            if pend_o[nb] is not None:
                pend_o[nb].wait()
            pend_g[nb] = pltpu.async_copy(
                table_hbm.at[idx_v.at[i + 1]], rows_v.at[nb], gsems[nb])
        pend_g[b].wait()
        pend_o[b] = pltpu.async_copy(
            rows_v.at[b], out_hbm.at[pl.ds(base + i * CHUNK, CHUNK)], osems[b])
    pend_o[0].wait()
    pend_o[1].wait()


def kernel(inputs, table):
    tpad = jnp.pad(table, ((0, 0), (0, PDIM - DIM)))
    idx = inputs.reshape(NW, NCHUNK, CHUNK).astype(jnp.int32)
    out = _gather(tpad, idx)
    return out[:, :DIM].reshape(B, L, DIM)


# speedup vs baseline: 1.4357x; 1.3215x over previous
import functools

import jax
import jax.numpy as jnp
from jax import lax
from jax.experimental import pallas as pl
from jax.experimental.pallas import tpu as pltpu
from jax.experimental.pallas import tpu_sc as plsc

VOCAB = 1000000
DIM = 64
B = 4096
L = 50
TOTAL = B * L

_info = plsc.get_sparse_core_info()
NC = _info.num_cores
NS = _info.num_subcores
NW = NC * NS
PER_W = TOTAL // NW
CHUNK = 320
NCHUNK = PER_W // CHUNK

_mesh = plsc.VectorSubcoreMesh(core_axis_name="c", subcore_axis_name="s")


@functools.partial(
    pl.kernel,
    mesh=_mesh,
    out_type=jax.ShapeDtypeStruct((TOTAL, DIM), jnp.float32),
    scratch_types=[
        pltpu.VMEM((PER_W,), jnp.int32),
        pltpu.VMEM((2, CHUNK, DIM), jnp.float32),
        pltpu.SemaphoreType.DMA,
        pltpu.SemaphoreType.DMA,
        pltpu.SemaphoreType.DMA,
    ],
)
def _gather(table_hbm, idx_hbm, out_hbm, idx_v, rows_v, gsem, osem0, osem1):
    wid = lax.axis_index("s") * NC + lax.axis_index("c")
    base = wid * PER_W
    osems = (osem0, osem1)
    pltpu.sync_copy(idx_hbm.at[wid], idx_v)

    def fire_chunk(i, bsel):
        def grp_body(k, carry):
            rvec = idx_v[pl.ds(i * CHUNK + k * 16, 16)]
            for l in range(16):
                pltpu.async_copy(
                    table_hbm.at[pl.ds(rvec[l], 1)],
                    rows_v.at[bsel, pl.ds(k * 16 + l, 1)],
                    gsem)
            return carry

        lax.fori_loop(0, CHUNK // 16, grp_body, 0)

    def drain_chunk(bsel):
        pltpu.make_async_copy(
            table_hbm.at[pl.ds(0, CHUNK)], rows_v.at[bsel], gsem).wait()

    pend_o = [None, None]
    fire_chunk(0, 0)
    for i in range(NCHUNK):
        b = i % 2
        nb = (i + 1) % 2
        drain_chunk(b)
        if i + 1 < NCHUNK:
            if pend_o[nb] is not None:
                pend_o[nb].wait()
            fire_chunk(i + 1, nb)
        pend_o[b] = pltpu.async_copy(
            rows_v.at[b], out_hbm.at[pl.ds(base + i * CHUNK, CHUNK)], osems[b])
    pend_o[0].wait()
    pend_o[1].wait()


def kernel(inputs, table):
    idx = inputs.reshape(NW, PER_W).astype(jnp.int32)
    out = _gather(table, idx)
    return out.reshape(B, L, DIM)


# speedup vs baseline: 1.6354x; 1.1391x over previous
import functools

import jax
import jax.numpy as jnp
from jax import lax
from jax.experimental import pallas as pl
from jax.experimental.pallas import tpu as pltpu
from jax.experimental.pallas import tpu_sc as plsc

VOCAB = 1000000
DIM = 64
B = 4096
L = 50
TOTAL = B * L

_info = plsc.get_sparse_core_info()
NC = _info.num_cores
NS = _info.num_subcores
NW = NC * NS
PER_W = TOTAL // NW
BATCH_W = B // NW
CB = 8
CHUNK = CB * L
NCHUNK = PER_W // CHUNK

_mesh = plsc.VectorSubcoreMesh(core_axis_name="c", subcore_axis_name="s")


@functools.partial(
    pl.kernel,
    mesh=_mesh,
    out_type=jax.ShapeDtypeStruct((B, L, DIM), jnp.float32),
    scratch_types=[
        pltpu.VMEM((PER_W,), jnp.int32),
        pltpu.VMEM((2, CHUNK, DIM), jnp.float32),
        pltpu.SemaphoreType.DMA,
        pltpu.SemaphoreType.DMA,
        pltpu.SemaphoreType.DMA,
    ],
)
def _gather(table_hbm, idx_hbm, out_hbm, idx_v, rows_v, gsem, osem0, osem1):
    wid = lax.axis_index("s") * NC + lax.axis_index("c")
    base_b = wid * BATCH_W
    osems = (osem0, osem1)
    pltpu.sync_copy(idx_hbm.at[wid], idx_v)

    def fire_chunk(i, bsel):
        def grp_body(k, carry):
            rvec = idx_v[pl.ds(i * CHUNK + k * 16, 16)]
            for l in range(16):
                pltpu.async_copy(
                    table_hbm.at[pl.ds(rvec[l], 1)],
                    rows_v.at[bsel, pl.ds(k * 16 + l, 1)],
                    gsem)
            return carry

        lax.fori_loop(0, CHUNK // 16, grp_body, 0)

    def drain_chunk(bsel):
        pltpu.make_async_copy(
            table_hbm.at[pl.ds(0, CHUNK)], rows_v.at[bsel], gsem).wait()

    pend_o = [None, None]
    fire_chunk(0, 0)
    for i in range(NCHUNK):
        b = i % 2
        nb = (i + 1) % 2
        drain_chunk(b)
        if i + 1 < NCHUNK:
            if pend_o[nb] is not None:
                for cp in pend_o[nb]:
                    cp.wait()
            fire_chunk(i + 1, nb)
        pend_o[b] = [
            pltpu.async_copy(
                rows_v.at[b, pl.ds(q * L, L)],
                out_hbm.at[base_b + i * CB + q],
                osems[b])
            for q in range(CB)
        ]
    for lst in pend_o:
        for cp in lst:
            cp.wait()


def kernel(inputs, table):
    idx = inputs.reshape(NW, PER_W).astype(jnp.int32)
    return _gather(table, idx)
